# baseline (speedup 1.0000x reference)
import numpy as np
import jax
import jax.numpy as jnp
from jax.experimental import pallas as pl
from jax.experimental.pallas import tpu as pltpu

_P = 0.5
_MAX_DROP = 8


def _drop_indices():
    rng = np.random.RandomState(1)
    if not (rng.rand() < _P):
        return np.zeros((0,), np.int32)
    num_drop = int(rng.randint(1, _MAX_DROP + 1))
    return np.sort(rng.permutation(96)[:num_drop].astype(np.int32))


_DROP = tuple(int(i) for i in _drop_indices())

_B, _C, _H, _W = 16, 96, 224, 224
_HW = _H * _W
_ROWS = _B * _C
_BLK_ROWS = 16


def _body(in_ref, out_ref):
    row0 = pl.program_id(0) * _BLK_ROWS
    r = jax.lax.broadcasted_iota(jnp.int32, (_BLK_ROWS, 1), 0) + row0
    c = r % _C
    keep = jnp.ones((_BLK_ROWS, 1), jnp.bool_)
    for d in _DROP:
        keep = jnp.logical_and(keep, c != d)
    out_ref[...] = jnp.where(keep, in_ref[...], 0.0)


def kernel(image):
    flat = image.reshape(_ROWS, _HW)
    out = pl.pallas_call(
        _body,
        grid=(_ROWS // _BLK_ROWS,),
        in_specs=[pl.BlockSpec((_BLK_ROWS, _HW), lambda i: (i, 0))],
        out_specs=pl.BlockSpec((_BLK_ROWS, _HW), lambda i: (i, 0)),
        out_shape=jax.ShapeDtypeStruct((_ROWS, _HW), jnp.float32),
        compiler_params=pltpu.CompilerParams(
            dimension_semantics=("arbitrary",),
        ),
    )(flat)
    return out.reshape(_B, _C, _H, _W)


# speedup vs baseline: 1.0039x; 1.0039x over previous
import numpy as np
import jax
import jax.numpy as jnp
from jax.experimental import pallas as pl
from jax.experimental.pallas import tpu as pltpu

_P = 0.5
_MAX_DROP = 8


def _drop_indices():
    rng = np.random.RandomState(1)
    if not (rng.rand() < _P):
        return np.zeros((0,), np.int32)
    num_drop = int(rng.randint(1, _MAX_DROP + 1))
    return np.sort(rng.permutation(96)[:num_drop].astype(np.int32))


_DROP = tuple(int(i) for i in _drop_indices())

_B, _C, _H, _W = 16, 96, 224, 224
_HW = _H * _W
_ROWS = _B * _C
_BLK_ROWS = 48


def _body(in_ref, out_ref):
    row0 = pl.program_id(0) * _BLK_ROWS
    r = jax.lax.broadcasted_iota(jnp.int32, (_BLK_ROWS, 1), 0) + row0
    c = r % _C
    keep = jnp.ones((_BLK_ROWS, 1), jnp.bool_)
    for d in _DROP:
        keep = jnp.logical_and(keep, c != d)
    out_ref[...] = jnp.where(keep, in_ref[...], 0.0)


def kernel(image):
    flat = image.reshape(_ROWS, _HW)
    out = pl.pallas_call(
        _body,
        grid=(_ROWS // _BLK_ROWS,),
        in_specs=[pl.BlockSpec((_BLK_ROWS, _HW), lambda i: (i, 0))],
        out_specs=pl.BlockSpec((_BLK_ROWS, _HW), lambda i: (i, 0)),
        out_shape=jax.ShapeDtypeStruct((_ROWS, _HW), jnp.float32),
        compiler_params=pltpu.CompilerParams(
            dimension_semantics=("arbitrary",),
        ),
    )(flat)
    return out.reshape(_B, _C, _H, _W)


# speedup vs baseline: 3.3957x; 3.3825x over previous
import numpy as np
import jax
import jax.numpy as jnp
from jax.experimental import pallas as pl
from jax.experimental.pallas import tpu as pltpu

_P = 0.5
_MAX_DROP = 8


def _drop_indices():
    rng = np.random.RandomState(1)
    if not (rng.rand() < _P):
        return np.zeros((0,), np.int32)
    num_drop = int(rng.randint(1, _MAX_DROP + 1))
    return np.sort(rng.permutation(96)[:num_drop].astype(np.int32))


_DROP = tuple(int(i) for i in _drop_indices())

_B, _C, _H, _W = 16, 96, 224, 224
_BLK_C = 8


def _body(in_ref, out_ref):
    c0 = pl.program_id(1) * _BLK_C
    c = jax.lax.broadcasted_iota(jnp.int32, (1, _BLK_C, 1, 1), 1) + c0
    keep = jnp.ones((1, _BLK_C, 1, 1), jnp.bool_)
    for d in _DROP:
        keep = jnp.logical_and(keep, c != d)
    out_ref[...] = jnp.where(keep, in_ref[...], 0.0)


def kernel(image):
    return pl.pallas_call(
        _body,
        grid=(_B, _C // _BLK_C),
        in_specs=[pl.BlockSpec((1, _BLK_C, _H, _W), lambda i, j: (i, j, 0, 0))],
        out_specs=pl.BlockSpec((1, _BLK_C, _H, _W), lambda i, j: (i, j, 0, 0)),
        out_shape=jax.ShapeDtypeStruct((_B, _C, _H, _W), jnp.float32),
        compiler_params=pltpu.CompilerParams(
            dimension_semantics=("arbitrary", "arbitrary"),
        ),
    )(image)


# speedup vs baseline: 3.8832x; 1.1436x over previous
import numpy as np
import jax
import jax.numpy as jnp
from jax.experimental import pallas as pl
from jax.experimental.pallas import tpu as pltpu

_P = 0.5
_MAX_DROP = 8


def _drop_indices():
    rng = np.random.RandomState(1)
    if not (rng.rand() < _P):
        return np.zeros((0,), np.int32)
    num_drop = int(rng.randint(1, _MAX_DROP + 1))
    return np.sort(rng.permutation(96)[:num_drop].astype(np.int32))


_DROP = tuple(int(i) for i in _drop_indices())

_B, _C, _H, _W = 16, 96, 224, 224
_BLK_C = 32


def _body(in_ref, out_ref):
    c0 = pl.program_id(1) * _BLK_C
    c = jax.lax.broadcasted_iota(jnp.int32, (1, _BLK_C, 1, 1), 1) + c0
    keep = jnp.ones((1, _BLK_C, 1, 1), jnp.bool_)
    for d in _DROP:
        keep = jnp.logical_and(keep, c != d)
    out_ref[...] = jnp.where(keep, in_ref[...], 0.0)


def kernel(image):
    return pl.pallas_call(
        _body,
        grid=(_B, _C // _BLK_C),
        in_specs=[pl.BlockSpec((1, _BLK_C, _H, _W), lambda i, j: (i, j, 0, 0))],
        out_specs=pl.BlockSpec((1, _BLK_C, _H, _W), lambda i, j: (i, j, 0, 0)),
        out_shape=jax.ShapeDtypeStruct((_B, _C, _H, _W), jnp.float32),
        compiler_params=pltpu.CompilerParams(
            dimension_semantics=("arbitrary", "arbitrary"),
        ),
    )(image)


# speedup vs baseline: 3.9050x; 1.0056x over previous
import numpy as np
import jax
import jax.numpy as jnp
from jax.experimental import pallas as pl
from jax.experimental.pallas import tpu as pltpu

_P = 0.5
_MAX_DROP = 8


def _drop_indices():
    rng = np.random.RandomState(1)
    if not (rng.rand() < _P):
        return np.zeros((0,), np.int32)
    num_drop = int(rng.randint(1, _MAX_DROP + 1))
    return np.sort(rng.permutation(96)[:num_drop].astype(np.int32))


_DROP = tuple(int(i) for i in _drop_indices())

_B, _C, _H, _W = 16, 96, 224, 224
_BLK_C = 48


def _body(in_ref, out_ref):
    c0 = pl.program_id(1) * _BLK_C
    c = jax.lax.broadcasted_iota(jnp.int32, (1, _BLK_C, 1, 1), 1) + c0
    keep = jnp.ones((1, _BLK_C, 1, 1), jnp.bool_)
    for d in _DROP:
        keep = jnp.logical_and(keep, c != d)
    out_ref[...] = jnp.where(keep, in_ref[...], 0.0)


def kernel(image):
    return pl.pallas_call(
        _body,
        grid=(_B, _C // _BLK_C),
        in_specs=[pl.BlockSpec((1, _BLK_C, _H, _W), lambda i, j: (i, j, 0, 0))],
        out_specs=pl.BlockSpec((1, _BLK_C, _H, _W), lambda i, j: (i, j, 0, 0)),
        out_shape=jax.ShapeDtypeStruct((_B, _C, _H, _W), jnp.float32),
        compiler_params=pltpu.CompilerParams(
            dimension_semantics=("arbitrary", "arbitrary"),
        ),
    )(image)
